# baseline (speedup 1.0000x reference)
import functools

import jax
import jax.numpy as jnp
from jax import lax
from jax.experimental import pallas as pl
from jax.experimental.pallas import tpu as pltpu
from jax.experimental.pallas import tpu_sc as plsc

NC = 2
NS = 16
LN = 16


def _pick_batch(ew: int) -> int:
    for b in (128, 112, 96, 80, 64, 48, 32, 16):
        if ew % b == 0:
            return b
    raise ValueError(f"per-subcore edge count {ew} not a multiple of 16")


def _mesh():
    return plsc.VectorSubcoreMesh(core_axis_name="c", subcore_axis_name="s",
                                  num_cores=NC, num_subcores=NS)


def _sc_prep(src, dst, etype, n):
    e = dst.shape[0]
    nw = NC * NS
    ew = e // nw

    @functools.partial(
        pl.kernel,
        out_type=(
            jax.ShapeDtypeStruct((e,), jnp.int32),
            jax.ShapeDtypeStruct((e,), jnp.int32),
        ),
        mesh=_mesh(),
        scratch_types=[
            pltpu.VMEM((ew,), jnp.int32),
            pltpu.VMEM((ew,), jnp.int32),
            pltpu.VMEM((ew,), jnp.int32),
        ],
    )
    def prep_kernel(src_hbm, dst_hbm, type_hbm, eidx_hbm, nidx_hbm,
                    src_v, dst_v, type_v):
        c = lax.axis_index("c")
        s = lax.axis_index("s")
        wid = s * NC + c
        base0 = wid * ew
        pltpu.sync_copy(src_hbm.at[pl.ds(base0, ew)], src_v)
        pltpu.sync_copy(dst_hbm.at[pl.ds(base0, ew)], dst_v)
        pltpu.sync_copy(type_hbm.at[pl.ds(base0, ew)], type_v)

        def pre(i, carry):
            sl = pl.ds(i * LN, LN)
            tt = type_v[sl]
            src_v[sl] = tt * n + src_v[sl]
            dst_v[sl] = dst_v[sl] * 128 + tt
            return carry
        lax.fori_loop(0, ew // LN, pre, None)
        pltpu.sync_copy(src_v, eidx_hbm.at[pl.ds(base0, ew)])
        pltpu.sync_copy(dst_v, nidx_hbm.at[pl.ds(base0, ew)])

    return prep_kernel(src, dst, etype)


def _sc_agg(h_tab, norm, eidx, nidx, dst, n, d, scale=True):
    e = dst.shape[0]
    nw = NC * NS
    ew = e // nw
    bsz = _pick_batch(ew)
    nb = ew // bsz
    rows_per_tile = n // NS
    nbuf = 4
    assert nb >= nbuf + 1

    scratch = [
        [pltpu.VMEM((bsz,), jnp.int32) for _ in range(nbuf)],
        [pltpu.VMEM((bsz,), jnp.int32) for _ in range(nbuf)],
        [pltpu.VMEM((bsz, d), jnp.float32) for _ in range(nbuf)],
        pltpu.VMEM_SHARED((n, d), jnp.float32),
        [pltpu.SemaphoreType.DMA for _ in range(nbuf)],
        [pltpu.SemaphoreType.DMA for _ in range(nbuf)],
        [pltpu.SemaphoreType.DMA for _ in range(nbuf)],
    ]
    if scale:
        scratch += [
            [pltpu.VMEM((bsz,), jnp.int32) for _ in range(nbuf)],
            [pltpu.VMEM((bsz,), jnp.float32) for _ in range(nbuf)],
            [pltpu.SemaphoreType.DMA for _ in range(nbuf)],
        ]

    @functools.partial(
        pl.kernel,
        out_type=jax.ShapeDtypeStruct((NC, n, d), jnp.float32),
        mesh=_mesh(),
        scratch_types=scratch,
    )
    def agg_kernel(*refs):
        if scale:
            (h_hbm, nrm_hbm, eidx_hbm, nidx_hbm, dst_hbm, out_hbm,
             eix, dix, rows, acc_sh, si, sg, ss, nix, wgt, sn) = refs
        else:
            (h_hbm, eidx_hbm, dst_hbm, out_hbm,
             eix, dix, rows, acc_sh, si, sg, ss) = refs
        c = lax.axis_index("c")
        s = lax.axis_index("s")
        wid = s * NC + c

        def zfill(i, carry):
            row = i // (d // LN)
            col = i % (d // LN)
            rows[0][row, pl.ds(col * LN, LN)] = jnp.zeros((LN,), jnp.float32)
            return carry
        lax.fori_loop(0, bsz * (d // LN), zfill, None)

        def zcopy(i, carry):
            pltpu.sync_copy(
                rows[0], acc_sh.at[pl.ds(s * rows_per_tile + i * bsz, bsz)])
            return carry
        lax.fori_loop(0, rows_per_tile // bsz, zcopy, None)
        ztail = rows_per_tile % bsz
        if ztail:
            pltpu.sync_copy(
                rows[0].at[pl.ds(0, ztail)],
                acc_sh.at[pl.ds(s * rows_per_tile
                                + (rows_per_tile // bsz) * bsz, ztail)])
        plsc.subcore_barrier()

        base0 = wid * ew

        def issue_idx(i, k):
            pltpu.async_copy(eidx_hbm.at[pl.ds(base0 + i * bsz, bsz)],
                             eix[k], si[k])
            if scale:
                pltpu.async_copy(nidx_hbm.at[pl.ds(base0 + i * bsz, bsz)],
                                 nix[k], si[k])
            pltpu.async_copy(dst_hbm.at[pl.ds(base0 + i * bsz, bsz)],
                             dix[k], si[k])

        def wait_idx(i, k):
            pltpu.make_async_copy(eidx_hbm.at[pl.ds(base0 + i * bsz, bsz)],
                                  eix[k], si[k]).wait()
            if scale:
                pltpu.make_async_copy(nidx_hbm.at[pl.ds(base0 + i * bsz, bsz)],
                                      nix[k], si[k]).wait()
            pltpu.make_async_copy(dst_hbm.at[pl.ds(base0 + i * bsz, bsz)],
                                  dix[k], si[k]).wait()

        def issue_gath(k):
            if scale:
                pltpu.async_copy(nrm_hbm.at[nix[k]], wgt[k], sn[k])
            pltpu.async_copy(h_hbm.at[eix[k]], rows[k], sg[k])

        def wait_gath(k):
            if scale:
                pltpu.make_async_copy(nrm_hbm.at[nix[k]], wgt[k],
                                      sn[k]).wait()
            pltpu.make_async_copy(h_hbm.at[eix[k]], rows[k], sg[k]).wait()

        def wait_scat(k):
            pltpu.make_async_copy(rows[k], acc_sh.at[dix[k]], ss[k]).wait()

        def step(i, k, wait_prev, guard_issue):
            k1 = (k + 1) % nbuf
            k2 = (k + 2) % nbuf
            if wait_prev:
                wait_scat(k2)
            if guard_issue:
                @pl.when(i + 2 < nb)
                def _issue_idx_next():
                    issue_idx(i + 2, k2)

                @pl.when(i + 1 < nb)
                def _advance_gath():
                    wait_idx(i + 1, k1)
                    issue_gath(k1)
            else:
                issue_idx(i + 2, k2)
                wait_idx(i + 1, k1)
                issue_gath(k1)
            wait_gath(k)

            if scale:
                def scale_rows(kk, carry2):
                    wv = wgt[k][pl.ds(kk * LN, LN)]
                    for jj in range(LN):
                        j = kk * LN + jj
                        wb = jnp.full((LN,), wv[jj], jnp.float32)
                        for cc in range(d // LN):
                            sl2 = pl.ds(cc * LN, LN)
                            rows[k][j, sl2] = rows[k][j, sl2] * wb
                    return carry2
                lax.fori_loop(0, bsz // LN, scale_rows, None)
            pltpu.async_copy(rows[k], acc_sh.at[dix[k]], ss[k], add=True)

        issue_idx(0, 0)
        issue_idx(1, 1)
        wait_idx(0, 0)
        issue_gath(0)
        step(0, 0, wait_prev=False, guard_issue=False)
        step(1, 1, wait_prev=False, guard_issue=False)
        quads = (nb - 2) // nbuf

        def quad(g, carry):
            i0 = 2 + g * nbuf
            for q in range(nbuf):
                step(i0 + q, (2 + q) % nbuf, wait_prev=True, guard_issue=True)
            return carry
        lax.fori_loop(0, quads, quad, None)
        for i in range(2 + quads * nbuf, nb):
            step(i, i % nbuf, wait_prev=True, guard_issue=True)
        wait_scat((nb - 2) % nbuf)
        wait_scat((nb - 1) % nbuf)

        plsc.subcore_barrier()
        g_per = (n // 8) // NS
        rem = (n // 8) - g_per * NS
        row0 = s * (g_per * 8)
        pltpu.sync_copy(acc_sh.at[pl.ds(row0, g_per * 8)],
                        out_hbm.at[c, pl.ds(row0, g_per * 8)])
        if rem:
            @pl.when(s == NS - 1)
            def _tail_copy():
                r0 = NS * g_per * 8
                pltpu.sync_copy(acc_sh.at[pl.ds(r0, rem * 8)],
                                out_hbm.at[c, pl.ds(r0, rem * 8)])

    if scale:
        return agg_kernel(h_tab, norm, eidx, nidx, dst)
    return agg_kernel(h_tab, eidx, dst)


def _rows_block(n):
    for b in (1000, 2000, 500, 200, 1024, 512, 256, 128):
        if n % b == 0:
            return b
    return n


def _relmat_body(x_ref, w_ref, o_ref):
    o_ref[...] = jnp.dot(x_ref[...], w_ref[0],
                         preferred_element_type=jnp.float32)


def _tc_relmat(xin, w_rel):
    n, d_in = xin.shape
    r, _, d_out = w_rel.shape
    blk = _rows_block(n)
    nbk = n // blk
    return pl.pallas_call(
        _relmat_body,
        grid=(r, nbk),
        in_specs=[
            pl.BlockSpec((blk, d_in), lambda ri, i: (i, 0)),
            pl.BlockSpec((1, d_in, d_out), lambda ri, i: (ri, 0, 0)),
        ],
        out_specs=pl.BlockSpec((blk, d_out), lambda ri, i: (ri * nbk + i, 0)),
        out_shape=jax.ShapeDtypeStruct((r * n, d_out), jnp.float32),
    )(xin, w_rel)


def _norm_body(c_ref, o_ref):
    o_ref[...] = 1.0 / jnp.maximum(c_ref[0] + c_ref[1], 1.0)


def _tc_norm(cnt_part):
    nc, n, d = cnt_part.shape
    blk = _rows_block(n)
    nbk = n // blk
    out = pl.pallas_call(
        _norm_body,
        grid=(nbk,),
        in_specs=[pl.BlockSpec((nc, blk, d), lambda i: (0, i, 0))],
        out_specs=pl.BlockSpec((blk, d), lambda i: (i, 0)),
        out_shape=jax.ShapeDtypeStruct((n, d), jnp.float32),
    )(cnt_part)
    return out.reshape(n * d)


def _combine_body_relu(agg_ref, x_ref, w_ref, b_ref, o_ref):
    v = (agg_ref[0] + agg_ref[1] + b_ref[...]
         + jnp.dot(x_ref[...], w_ref[...], preferred_element_type=jnp.float32))
    o_ref[...] = jnp.maximum(v, 0.0)


def _combine_body(agg_ref, x_ref, w_ref, b_ref, o_ref):
    o_ref[...] = (agg_ref[0] + agg_ref[1] + b_ref[...]
                  + jnp.dot(x_ref[...], w_ref[...],
                            preferred_element_type=jnp.float32))


def _tc_combine(agg, xin, w_root, b, relu):
    n, d_in = xin.shape
    d_out = w_root.shape[1]
    blk = _rows_block(n)
    nbk = n // blk
    body = _combine_body_relu if relu else _combine_body
    return pl.pallas_call(
        body,
        grid=(nbk,),
        in_specs=[
            pl.BlockSpec((NC, blk, d_out), lambda i: (0, i, 0)),
            pl.BlockSpec((blk, d_in), lambda i: (i, 0)),
            pl.BlockSpec((d_in, d_out), lambda i: (0, 0)),
            pl.BlockSpec((1, d_out), lambda i: (0, 0)),
        ],
        out_specs=pl.BlockSpec((blk, d_out), lambda i: (i, 0)),
        out_shape=jax.ShapeDtypeStruct((n, d_out), jnp.float32),
    )(agg, xin, w_root, b.reshape(1, d_out))


def kernel(x, edge_index, edge_type, W_rel1, W_root1, b1, W_rel2, W_root2, b2):
    n, _ = x.shape
    r = W_rel1.shape[0]
    d_hid = W_rel1.shape[2]
    d_out = W_rel2.shape[2]
    src = edge_index[0]
    dst = edge_index[1]

    eye_tab = jnp.eye(r, 128, dtype=jnp.float32)
    eidx, nidx = _sc_prep(src, dst, edge_type, n)
    cnt_part = _sc_agg(eye_tab, None, edge_type, None, dst, n, 128,
                       scale=False)
    norm = _tc_norm(cnt_part)

    h_tab1 = _tc_relmat(x, W_rel1)
    agg1 = _sc_agg(h_tab1, norm, eidx, nidx, dst, n, d_hid)
    h1 = _tc_combine(agg1, x, W_root1, b1, relu=True)

    h_tab2 = _tc_relmat(h1, W_rel2)
    agg2 = _sc_agg(h_tab2, norm, eidx, nidx, dst, n, d_out)
    out = _tc_combine(agg2, h1, W_root2, b2, relu=False)
    return out


# speedup vs baseline: 3.8698x; 3.8698x over previous
import functools

import jax
import jax.numpy as jnp
from jax import lax
from jax.experimental import pallas as pl
from jax.experimental.pallas import tpu as pltpu
from jax.experimental.pallas import tpu_sc as plsc

NC = 2
NS = 16
LN = 16
EYE_REP = 512


def _pick_batch(ew: int) -> int:
    for b in (128, 112, 96, 80, 64, 48, 32, 16):
        if ew % b == 0:
            return b
    raise ValueError(f"per-subcore edge count {ew} not a multiple of 16")


def _mesh():
    return plsc.VectorSubcoreMesh(core_axis_name="c", subcore_axis_name="s",
                                  num_cores=NC, num_subcores=NS)


def _sc_prep(src, dst, etype, n):
    e = dst.shape[0]
    nw = NC * NS
    ew = e // nw

    @functools.partial(
        pl.kernel,
        out_type=(
            jax.ShapeDtypeStruct((e,), jnp.int32),
            jax.ShapeDtypeStruct((e,), jnp.int32),
            jax.ShapeDtypeStruct((e,), jnp.int32),
        ),
        mesh=_mesh(),
        scratch_types=[
            pltpu.VMEM((ew,), jnp.int32),
            pltpu.VMEM((ew,), jnp.int32),
            pltpu.VMEM((ew,), jnp.int32),
        ],
    )
    def prep_kernel(src_hbm, dst_hbm, type_hbm, eidx_hbm, nidx_hbm, cidx_hbm,
                    src_v, dst_v, type_v):
        c = lax.axis_index("c")
        s = lax.axis_index("s")
        wid = s * NC + c
        base0 = wid * ew
        pltpu.sync_copy(src_hbm.at[pl.ds(base0, ew)], src_v)
        pltpu.sync_copy(dst_hbm.at[pl.ds(base0, ew)], dst_v)
        pltpu.sync_copy(type_hbm.at[pl.ds(base0, ew)], type_v)

        lanes = jnp.arange(LN, dtype=jnp.int32)

        def pre(i, carry):
            sl = pl.ds(i * LN, LN)
            tt = type_v[sl]
            src_v[sl] = tt * n + src_v[sl]
            dst_v[sl] = dst_v[sl] * 128 + tt
            rep = (i * LN + lanes) & (EYE_REP - 1)
            type_v[sl] = tt + 8 * rep
            return carry
        lax.fori_loop(0, ew // LN, pre, None)
        pltpu.sync_copy(src_v, eidx_hbm.at[pl.ds(base0, ew)])
        pltpu.sync_copy(dst_v, nidx_hbm.at[pl.ds(base0, ew)])
        pltpu.sync_copy(type_v, cidx_hbm.at[pl.ds(base0, ew)])

    return prep_kernel(src, dst, etype)


def _sc_agg(h_tab, norm, eidx, nidx, dst, n, d, scale=True):
    e = dst.shape[0]
    nw = NC * NS
    ew = e // nw
    bsz = _pick_batch(ew)
    nb = ew // bsz
    rows_per_tile = n // NS
    nbuf = 4
    assert nb >= nbuf + 1

    scratch = [
        [pltpu.VMEM((bsz,), jnp.int32) for _ in range(nbuf)],
        [pltpu.VMEM((bsz,), jnp.int32) for _ in range(nbuf)],
        [pltpu.VMEM((bsz, d), jnp.float32) for _ in range(nbuf)],
        pltpu.VMEM_SHARED((n, d), jnp.float32),
        [pltpu.SemaphoreType.DMA for _ in range(nbuf)],
        [pltpu.SemaphoreType.DMA for _ in range(nbuf)],
        [pltpu.SemaphoreType.DMA for _ in range(nbuf)],
    ]
    if scale:
        scratch += [
            [pltpu.VMEM((bsz,), jnp.int32) for _ in range(nbuf)],
            [pltpu.VMEM((bsz,), jnp.float32) for _ in range(nbuf)],
            [pltpu.SemaphoreType.DMA for _ in range(nbuf)],
        ]

    @functools.partial(
        pl.kernel,
        out_type=jax.ShapeDtypeStruct((NC, n, d), jnp.float32),
        mesh=_mesh(),
        scratch_types=scratch,
    )
    def agg_kernel(*refs):
        if scale:
            (h_hbm, nrm_hbm, eidx_hbm, nidx_hbm, dst_hbm, out_hbm,
             eix, dix, rows, acc_sh, si, sg, ss, nix, wgt, sn) = refs
        else:
            (h_hbm, eidx_hbm, dst_hbm, out_hbm,
             eix, dix, rows, acc_sh, si, sg, ss) = refs
        c = lax.axis_index("c")
        s = lax.axis_index("s")
        wid = s * NC + c

        def zfill(i, carry):
            row = i // (d // LN)
            col = i % (d // LN)
            rows[0][row, pl.ds(col * LN, LN)] = jnp.zeros((LN,), jnp.float32)
            return carry
        lax.fori_loop(0, bsz * (d // LN), zfill, None)

        def zcopy(i, carry):
            pltpu.sync_copy(
                rows[0], acc_sh.at[pl.ds(s * rows_per_tile + i * bsz, bsz)])
            return carry
        lax.fori_loop(0, rows_per_tile // bsz, zcopy, None)
        ztail = rows_per_tile % bsz
        if ztail:
            pltpu.sync_copy(
                rows[0].at[pl.ds(0, ztail)],
                acc_sh.at[pl.ds(s * rows_per_tile
                                + (rows_per_tile // bsz) * bsz, ztail)])
        plsc.subcore_barrier()

        base0 = wid * ew

        def issue_idx(i, k):
            pltpu.async_copy(eidx_hbm.at[pl.ds(base0 + i * bsz, bsz)],
                             eix[k], si[k])
            if scale:
                pltpu.async_copy(nidx_hbm.at[pl.ds(base0 + i * bsz, bsz)],
                                 nix[k], si[k])
            pltpu.async_copy(dst_hbm.at[pl.ds(base0 + i * bsz, bsz)],
                             dix[k], si[k])

        def wait_idx(i, k):
            pltpu.make_async_copy(eidx_hbm.at[pl.ds(base0 + i * bsz, bsz)],
                                  eix[k], si[k]).wait()
            if scale:
                pltpu.make_async_copy(nidx_hbm.at[pl.ds(base0 + i * bsz, bsz)],
                                      nix[k], si[k]).wait()
            pltpu.make_async_copy(dst_hbm.at[pl.ds(base0 + i * bsz, bsz)],
                                  dix[k], si[k]).wait()

        def issue_gath(k):
            if scale:
                pltpu.async_copy(nrm_hbm.at[nix[k]], wgt[k], sn[k])
            pltpu.async_copy(h_hbm.at[eix[k]], rows[k], sg[k])

        def wait_gath(k):
            if scale:
                pltpu.make_async_copy(nrm_hbm.at[nix[k]], wgt[k],
                                      sn[k]).wait()
            pltpu.make_async_copy(h_hbm.at[eix[k]], rows[k], sg[k]).wait()

        def wait_scat(k):
            pltpu.make_async_copy(rows[k], acc_sh.at[dix[k]], ss[k]).wait()

        def step(i, k, wait_prev, guard_issue):
            k1 = (k + 1) % nbuf
            k2 = (k + 2) % nbuf
            if wait_prev:
                wait_scat(k2)
            if guard_issue:
                @pl.when(i + 2 < nb)
                def _issue_idx_next():
                    issue_idx(i + 2, k2)

                @pl.when(i + 1 < nb)
                def _advance_gath():
                    wait_idx(i + 1, k1)
                    issue_gath(k1)
            else:
                issue_idx(i + 2, k2)
                wait_idx(i + 1, k1)
                issue_gath(k1)
            wait_gath(k)

            if scale:
                def scale_rows(kk, carry2):
                    wv = wgt[k][pl.ds(kk * LN, LN)]
                    for jj in range(LN):
                        j = kk * LN + jj
                        wb = jnp.full((LN,), wv[jj], jnp.float32)
                        for cc in range(d // LN):
                            sl2 = pl.ds(cc * LN, LN)
                            rows[k][j, sl2] = rows[k][j, sl2] * wb
                    return carry2
                lax.fori_loop(0, bsz // LN, scale_rows, None)
            pltpu.async_copy(rows[k], acc_sh.at[dix[k]], ss[k], add=True)

        issue_idx(0, 0)
        issue_idx(1, 1)
        wait_idx(0, 0)
        issue_gath(0)
        step(0, 0, wait_prev=False, guard_issue=False)
        step(1, 1, wait_prev=False, guard_issue=False)
        quads = (nb - 2) // nbuf

        def quad(g, carry):
            i0 = 2 + g * nbuf
            for q in range(nbuf):
                step(i0 + q, (2 + q) % nbuf, wait_prev=True, guard_issue=True)
            return carry
        lax.fori_loop(0, quads, quad, None)
        for i in range(2 + quads * nbuf, nb):
            step(i, i % nbuf, wait_prev=True, guard_issue=True)
        wait_scat((nb - 2) % nbuf)
        wait_scat((nb - 1) % nbuf)

        plsc.subcore_barrier()
        g_per = (n // 8) // NS
        rem = (n // 8) - g_per * NS
        row0 = s * (g_per * 8)
        pltpu.sync_copy(acc_sh.at[pl.ds(row0, g_per * 8)],
                        out_hbm.at[c, pl.ds(row0, g_per * 8)])
        if rem:
            @pl.when(s == NS - 1)
            def _tail_copy():
                r0 = NS * g_per * 8
                pltpu.sync_copy(acc_sh.at[pl.ds(r0, rem * 8)],
                                out_hbm.at[c, pl.ds(r0, rem * 8)])

    if scale:
        return agg_kernel(h_tab, norm, eidx, nidx, dst)
    return agg_kernel(h_tab, eidx, dst)


def _rows_block(n):
    for b in (1000, 2000, 500, 200, 1024, 512, 256, 128):
        if n % b == 0:
            return b
    return n


def _relmat_body(x_ref, w_ref, o_ref):
    o_ref[...] = jnp.dot(x_ref[...], w_ref[0],
                         preferred_element_type=jnp.float32)


def _tc_relmat(xin, w_rel):
    n, d_in = xin.shape
    r, _, d_out = w_rel.shape
    blk = _rows_block(n)
    nbk = n // blk
    return pl.pallas_call(
        _relmat_body,
        grid=(r, nbk),
        in_specs=[
            pl.BlockSpec((blk, d_in), lambda ri, i: (i, 0)),
            pl.BlockSpec((1, d_in, d_out), lambda ri, i: (ri, 0, 0)),
        ],
        out_specs=pl.BlockSpec((blk, d_out), lambda ri, i: (ri * nbk + i, 0)),
        out_shape=jax.ShapeDtypeStruct((r * n, d_out), jnp.float32),
    )(xin, w_rel)


def _norm_body(c_ref, o_ref):
    o_ref[...] = 1.0 / jnp.maximum(c_ref[0] + c_ref[1], 1.0)


def _tc_norm(cnt_part):
    nc, n, d = cnt_part.shape
    blk = _rows_block(n)
    nbk = n // blk
    out = pl.pallas_call(
        _norm_body,
        grid=(nbk,),
        in_specs=[pl.BlockSpec((nc, blk, d), lambda i: (0, i, 0))],
        out_specs=pl.BlockSpec((blk, d), lambda i: (i, 0)),
        out_shape=jax.ShapeDtypeStruct((n, d), jnp.float32),
    )(cnt_part)
    return out.reshape(n * d)


def _combine_body_relu(agg_ref, x_ref, w_ref, b_ref, o_ref):
    v = (agg_ref[0] + agg_ref[1] + b_ref[...]
         + jnp.dot(x_ref[...], w_ref[...], preferred_element_type=jnp.float32))
    o_ref[...] = jnp.maximum(v, 0.0)


def _combine_body(agg_ref, x_ref, w_ref, b_ref, o_ref):
    o_ref[...] = (agg_ref[0] + agg_ref[1] + b_ref[...]
                  + jnp.dot(x_ref[...], w_ref[...],
                            preferred_element_type=jnp.float32))


def _tc_combine(agg, xin, w_root, b, relu):
    n, d_in = xin.shape
    d_out = w_root.shape[1]
    blk = _rows_block(n)
    nbk = n // blk
    body = _combine_body_relu if relu else _combine_body
    return pl.pallas_call(
        body,
        grid=(nbk,),
        in_specs=[
            pl.BlockSpec((NC, blk, d_out), lambda i: (0, i, 0)),
            pl.BlockSpec((blk, d_in), lambda i: (i, 0)),
            pl.BlockSpec((d_in, d_out), lambda i: (0, 0)),
            pl.BlockSpec((1, d_out), lambda i: (0, 0)),
        ],
        out_specs=pl.BlockSpec((blk, d_out), lambda i: (i, 0)),
        out_shape=jax.ShapeDtypeStruct((n, d_out), jnp.float32),
    )(agg, xin, w_root, b.reshape(1, d_out))


def kernel(x, edge_index, edge_type, W_rel1, W_root1, b1, W_rel2, W_root2, b2):
    n, _ = x.shape
    r = W_rel1.shape[0]
    d_hid = W_rel1.shape[2]
    d_out = W_rel2.shape[2]
    src = edge_index[0]
    dst = edge_index[1]

    eye_tab = jnp.tile(jnp.eye(r, 128, dtype=jnp.float32), (EYE_REP, 1))
    eidx, nidx, cidx = _sc_prep(src, dst, edge_type, n)
    cnt_part = _sc_agg(eye_tab, None, cidx, None, dst, n, 128,
                       scale=False)
    norm = _tc_norm(cnt_part)

    h_tab1 = _tc_relmat(x, W_rel1)
    agg1 = _sc_agg(h_tab1, norm, eidx, nidx, dst, n, d_hid)
    h1 = _tc_combine(agg1, x, W_root1, b1, relu=True)

    h_tab2 = _tc_relmat(h1, W_rel2)
    agg2 = _sc_agg(h_tab2, norm, eidx, nidx, dst, n, d_out)
    out = _tc_combine(agg2, h1, W_root2, b2, relu=False)
    return out
